# baseline (speedup 1.0000x reference)
import functools

import jax
import jax.numpy as jnp
from jax import lax
from jax.experimental import pallas as pl
from jax.experimental.pallas import tpu as pltpu
from jax.experimental.pallas import tpu_sc as plsc

_NC = 2
_NS = 16
_SEG = 128


def _sc_gather_body(tbl_ref, tgt_ref, out_ref, tgt_v, idx_v, rows_v, sem, *, n):
    rpw = tgt_v.shape[0]
    nw = tgt_ref.shape[0] // rpw
    wid = lax.axis_index("s") * _NC + lax.axis_index("c")

    @pl.when(wid < nw)
    def _():
        base = wid * rpw
        pltpu.sync_copy(tgt_ref.at[pl.ds(base, rpw)], tgt_v)
        t = tgt_v[...]
        lane = lax.iota(jnp.int32, 16)
        flat = (lane + base) * n + t
        idx_v[...] = flat >> 7
        pltpu.async_copy(tbl_ref.at[idx_v], rows_v, sem).wait()
        pltpu.sync_copy(rows_v, out_ref.at[pl.ds(base, rpw)])


def _count_body(g_ref, tgt_ref, x_ref, out_ref, tacc, gt_acc, eq_acc,
                *, n, w, nb, b_rows):
    b = pl.program_id(0)
    tgt = tgt_ref[...]

    @pl.when(b == 0)
    def _():
        row = jax.lax.broadcasted_iota(jnp.int32, (b_rows, 1), 0)
        off = (row * n + tgt) & (_SEG - 1)
        colid = jax.lax.broadcasted_iota(jnp.int32, (b_rows, _SEG), 1)
        tacc[...] = jnp.sum(jnp.where(colid == off, g_ref[...], 0.0),
                            axis=1, keepdims=True)
        gt_acc[...] = jnp.zeros_like(gt_acc)
        eq_acc[...] = jnp.zeros_like(eq_acc)

    x = x_ref[...]
    col = jax.lax.broadcasted_iota(jnp.int32, x.shape, 1) + b * w
    t = tacc[...]
    valid = col < n
    gt = (x > t) & valid
    eq = (x == t) & (col < tgt)
    gt_acc[...] += jnp.sum(gt.astype(jnp.int32), axis=1, keepdims=True)
    eq_acc[...] += jnp.sum(eq.astype(jnp.int32), axis=1, keepdims=True)

    @pl.when(b == nb - 1)
    def _():
        rank = (1 + gt_acc[...] + eq_acc[...]).astype(jnp.float32)
        out_ref[0, 0] = jnp.sum(1.0 / rank) * (1.0 / b_rows)


@jax.jit
def kernel(logits, targets):
    if targets.ndim == 2:
        targets = jnp.squeeze(targets, axis=1)
    b_rows, n = logits.shape
    tgt = targets.astype(jnp.int32)

    rpw = 16
    tbl = logits.reshape(b_rows * n // _SEG, _SEG)
    sc_gather = pl.kernel(
        functools.partial(_sc_gather_body, n=n),
        out_type=jax.ShapeDtypeStruct((b_rows, _SEG), jnp.float32),
        mesh=plsc.VectorSubcoreMesh(core_axis_name="c", subcore_axis_name="s"),
        scratch_types=[
            pltpu.VMEM((rpw,), jnp.int32),
            pltpu.VMEM((rpw,), jnp.int32),
            pltpu.VMEM((rpw, _SEG), jnp.float32),
            pltpu.SemaphoreType.DMA,
        ],
    )
    segs = sc_gather(tbl, tgt)

    w = 8192
    nb = (n + w - 1) // w
    out = pl.pallas_call(
        functools.partial(_count_body, n=n, w=w, nb=nb, b_rows=b_rows),
        grid=(nb,),
        in_specs=[
            pl.BlockSpec((b_rows, _SEG), lambda b: (0, 0)),
            pl.BlockSpec((b_rows, 1), lambda b: (0, 0)),
            pl.BlockSpec((b_rows, w), lambda b: (0, b)),
        ],
        out_specs=pl.BlockSpec(memory_space=pltpu.SMEM),
        out_shape=jax.ShapeDtypeStruct((1, 1), jnp.float32),
        scratch_shapes=[
            pltpu.VMEM((b_rows, 1), jnp.float32),
            pltpu.VMEM((b_rows, 1), jnp.int32),
            pltpu.VMEM((b_rows, 1), jnp.int32),
        ],
    )(segs, tgt.reshape(b_rows, 1), logits)
    return out[0, 0]


# speedup vs baseline: 1.0992x; 1.0992x over previous
import functools

import jax
import jax.numpy as jnp
from jax.experimental import pallas as pl
from jax.experimental.pallas import tpu as pltpu

_SEG = 128


def _mrr_body(seg_src, tgt_s, tgt_ref, x_ref, out_ref, segs, tacc, gt_acc, eq_acc,
              sem, *, n, w, nb, b_rows):
    b = pl.program_id(0)
    tgt = tgt_ref[...]

    @pl.when(b == 0)
    def _():
        def issue(i, c):
            seg = (i * n + tgt_s[i]) >> 7
            pltpu.make_async_copy(seg_src.at[pl.ds(seg, 1)],
                                  segs.at[pl.ds(i, 1)], sem).start()
            return c

        jax.lax.fori_loop(0, b_rows, issue, 0)

        def drain(i, c):
            pltpu.make_async_copy(seg_src.at[pl.ds(0, 1)],
                                  segs.at[pl.ds(i, 1)], sem).wait()
            return c

        jax.lax.fori_loop(0, b_rows, drain, 0)

        row = jax.lax.broadcasted_iota(jnp.int32, (b_rows, 1), 0)
        off = (row * n + tgt) & (_SEG - 1)
        colid = jax.lax.broadcasted_iota(jnp.int32, (b_rows, _SEG), 1)
        tacc[...] = jnp.sum(jnp.where(colid == off, segs[...], 0.0),
                            axis=1, keepdims=True)
        gt_acc[...] = jnp.zeros_like(gt_acc)
        eq_acc[...] = jnp.zeros_like(eq_acc)

    x = x_ref[...]
    col = jax.lax.broadcasted_iota(jnp.int32, x.shape, 1) + b * w
    t = tacc[...]
    valid = col < n
    gt = (x > t) & valid
    eq = (x == t) & (col < tgt)
    gt_acc[...] += jnp.sum(gt.astype(jnp.int32), axis=1, keepdims=True)
    eq_acc[...] += jnp.sum(eq.astype(jnp.int32), axis=1, keepdims=True)

    @pl.when(b == nb - 1)
    def _():
        rank = (1 + gt_acc[...] + eq_acc[...]).astype(jnp.float32)
        out_ref[0, 0] = jnp.sum(1.0 / rank) * (1.0 / b_rows)


@jax.jit
def kernel(logits, targets):
    if targets.ndim == 2:
        targets = jnp.squeeze(targets, axis=1)
    b_rows, n = logits.shape
    tgt = targets.astype(jnp.int32)
    tbl = logits.reshape(b_rows * n // _SEG, _SEG)

    w = 8192
    nb = (n + w - 1) // w
    out = pl.pallas_call(
        functools.partial(_mrr_body, n=n, w=w, nb=nb, b_rows=b_rows),
        grid=(nb,),
        in_specs=[
            pl.BlockSpec(memory_space=pl.ANY),
            pl.BlockSpec(memory_space=pltpu.SMEM),
            pl.BlockSpec((b_rows, 1), lambda b: (0, 0)),
            pl.BlockSpec((b_rows, w), lambda b: (0, b)),
        ],
        out_specs=pl.BlockSpec(memory_space=pltpu.SMEM),
        out_shape=jax.ShapeDtypeStruct((1, 1), jnp.float32),
        scratch_shapes=[
            pltpu.VMEM((b_rows, _SEG), jnp.float32),
            pltpu.VMEM((b_rows, 1), jnp.float32),
            pltpu.VMEM((b_rows, 1), jnp.int32),
            pltpu.VMEM((b_rows, 1), jnp.int32),
            pltpu.SemaphoreType.DMA,
        ],
    )(tbl, tgt, tgt.reshape(b_rows, 1), logits)
    return out[0, 0]


# speedup vs baseline: 2.1559x; 1.9614x over previous
import functools

import jax
import jax.numpy as jnp
from jax.experimental import pallas as pl
from jax.experimental.pallas import tpu as pltpu

_SEG = 128
_SUB = 8


def _mrr_body(seg_src, tgt_s, tgt_ref, x_ref, out_ref, segs, tacc, gt_acc, eq_acc,
              sem, *, n, w, nb, b_rows):
    b = pl.program_id(0)
    tgt = tgt_ref[...]

    @pl.when(b == 0)
    def _():
        def issue(i, c):
            r0 = pl.multiple_of((i >> 3) << 3, _SUB)
            c0 = pl.multiple_of((tgt_s[i] >> 7) << 7, _SEG)
            pltpu.make_async_copy(
                seg_src.at[pl.ds(r0, _SUB), pl.ds(c0, _SEG)],
                segs.at[i], sem).start()
            return c

        jax.lax.fori_loop(0, b_rows, issue, 0)

        def drain(i, c):
            pltpu.make_async_copy(
                seg_src.at[pl.ds(0, _SUB), pl.ds(0, _SEG)],
                segs.at[i], sem).wait()
            return c

        jax.lax.fori_loop(0, b_rows, drain, 0)

        off = tgt & (_SEG - 1)
        rowph = jax.lax.broadcasted_iota(jnp.int32, (b_rows, 1), 0) & (_SUB - 1)
        colid = jax.lax.broadcasted_iota(jnp.int32, (b_rows, _SEG), 1)
        t = jnp.zeros((b_rows, 1), jnp.float32)
        for k in range(_SUB):
            sel = (colid == off) & (rowph == k)
            t = t + jnp.sum(jnp.where(sel, segs[:, k, :], 0.0),
                            axis=1, keepdims=True)
        tacc[...] = t
        gt_acc[...] = jnp.zeros_like(gt_acc)
        eq_acc[...] = jnp.zeros_like(eq_acc)

    x = x_ref[...]
    col = jax.lax.broadcasted_iota(jnp.int32, x.shape, 1) + b * w
    t = tacc[...]
    valid = col < n
    gt = (x > t) & valid
    eq = (x == t) & (col < tgt)
    gt_acc[...] += jnp.sum(gt.astype(jnp.int32), axis=1, keepdims=True)
    eq_acc[...] += jnp.sum(eq.astype(jnp.int32), axis=1, keepdims=True)

    @pl.when(b == nb - 1)
    def _():
        rank = (1 + gt_acc[...] + eq_acc[...]).astype(jnp.float32)
        out_ref[0, 0] = jnp.sum(1.0 / rank) * (1.0 / b_rows)


@jax.jit
def kernel(logits, targets):
    if targets.ndim == 2:
        targets = jnp.squeeze(targets, axis=1)
    b_rows, n = logits.shape
    tgt = targets.astype(jnp.int32)

    w = 8192
    nb = (n + w - 1) // w
    out = pl.pallas_call(
        functools.partial(_mrr_body, n=n, w=w, nb=nb, b_rows=b_rows),
        grid=(nb,),
        in_specs=[
            pl.BlockSpec(memory_space=pl.ANY),
            pl.BlockSpec(memory_space=pltpu.SMEM),
            pl.BlockSpec((b_rows, 1), lambda b: (0, 0)),
            pl.BlockSpec((b_rows, w), lambda b: (0, b)),
        ],
        out_specs=pl.BlockSpec(memory_space=pltpu.SMEM),
        out_shape=jax.ShapeDtypeStruct((1, 1), jnp.float32),
        scratch_shapes=[
            pltpu.VMEM((b_rows, _SUB, _SEG), jnp.float32),
            pltpu.VMEM((b_rows, 1), jnp.float32),
            pltpu.VMEM((b_rows, 1), jnp.int32),
            pltpu.VMEM((b_rows, 1), jnp.int32),
            pltpu.SemaphoreType.DMA,
        ],
    )(logits, tgt, tgt.reshape(b_rows, 1), logits)
    return out[0, 0]
